# baseline (speedup 1.0000x reference)
import functools

import jax
import jax.numpy as jnp
from jax import lax
from jax.experimental import pallas as pl
from jax.experimental.pallas import tpu as pltpu
from jax.experimental.pallas import tpu_sc as plsc

NUM_CLASSES = 1000
TABLE_ROWS = NUM_CLASSES + 1
HIDDEN = 128
BATCH = 16384
LOAD_ROWS = 64

NUM_CORES = 2
NUM_SUBCORES = 16
NUM_WORKERS = NUM_CORES * NUM_SUBCORES
B_PER_W = BATCH // NUM_WORKERS
IDX_CHUNK = 64
N_CHUNKS = B_PER_W // IDX_CHUNK

_mesh = plsc.VectorSubcoreMesh(core_axis_name="c", subcore_axis_name="s")


@functools.partial(
    pl.kernel,
    out_type=jax.ShapeDtypeStruct((BATCH, HIDDEN), jnp.float32),
    mesh=_mesh,
    scratch_types=[
        pltpu.VMEM((B_PER_W,), jnp.int32),
        pltpu.VMEM((B_PER_W, HIDDEN), jnp.float32),
        pltpu.VMEM_SHARED((TABLE_ROWS, HIDDEN), jnp.float32),
        pltpu.SemaphoreType.DMA((N_CHUNKS,)),
        pltpu.SemaphoreType.DMA,
        pltpu.SemaphoreType.DMA,
        pltpu.SemaphoreType.DMA,
    ],
)
def _sc_gather(
    table_hbm, labels_hbm, out_hbm, idx_v, rows_v, table_sp, gsem, osem, isem,
    tsem,
):
    sid = lax.axis_index("s")
    wid = sid * NUM_CORES + lax.axis_index("c")
    base = wid * B_PER_W
    idx_c = pltpu.async_copy(labels_hbm.at[pl.ds(base, B_PER_W)], idx_v, isem)
    slab = pl.multiple_of(
        lax.min(sid * LOAD_ROWS, (NUM_SUBCORES - 1) * LOAD_ROWS - 24), 8
    )
    tab_c = pltpu.async_copy(
        table_hbm.at[pl.ds(slab, LOAD_ROWS)],
        table_sp.at[pl.ds(slab, LOAD_ROWS)],
        tsem,
    )

    @pl.when(sid == 0)
    def _load_last_row():
        pltpu.sync_copy(
            table_hbm.at[pl.ds(TABLE_ROWS - 1, 1)],
            table_sp.at[pl.ds(TABLE_ROWS - 1, 1)],
        )

    idx_c.wait()
    tab_c.wait()
    plsc.subcore_barrier()
    gathers = []
    for j in range(N_CHUNKS):
        gathers.append(
            pltpu.async_copy(
                table_sp.at[idx_v.at[pl.ds(j * IDX_CHUNK, IDX_CHUNK)]],
                rows_v.at[pl.ds(j * IDX_CHUNK, IDX_CHUNK)],
                gsem.at[j],
            )
        )
    outs = []
    for j in range(N_CHUNKS):
        gathers[j].wait()
        outs.append(
            pltpu.async_copy(
                rows_v.at[pl.ds(j * IDX_CHUNK, IDX_CHUNK)],
                out_hbm.at[pl.ds(base + j * IDX_CHUNK, IDX_CHUNK)],
                osem,
            )
        )
    for c in outs:
        c.wait()


def kernel(labels, train, embedding_table):
    del train
    return _sc_gather(embedding_table, labels.astype(jnp.int32))


# speedup vs baseline: 1.0112x; 1.0112x over previous
import functools

import jax
import jax.numpy as jnp
from jax import lax
from jax.experimental import pallas as pl
from jax.experimental.pallas import tpu as pltpu
from jax.experimental.pallas import tpu_sc as plsc

NUM_CLASSES = 1000
TABLE_ROWS = NUM_CLASSES + 1
HIDDEN = 128
BATCH = 16384
LOAD_ROWS = 64

NUM_CORES = 2
NUM_SUBCORES = 16
NUM_WORKERS = NUM_CORES * NUM_SUBCORES
B_PER_W = BATCH // NUM_WORKERS
IDX_CHUNK = 64
N_CHUNKS = B_PER_W // IDX_CHUNK

_mesh = plsc.VectorSubcoreMesh(core_axis_name="c", subcore_axis_name="s")


@functools.partial(
    pl.kernel,
    out_type=jax.ShapeDtypeStruct((BATCH, HIDDEN), jnp.float32),
    mesh=_mesh,
    scratch_types=[
        pltpu.VMEM((B_PER_W,), jnp.int32),
        pltpu.VMEM((B_PER_W, HIDDEN), jnp.float32),
        pltpu.VMEM_SHARED((TABLE_ROWS, HIDDEN), jnp.float32),
        pltpu.SemaphoreType.DMA((N_CHUNKS,)),
        pltpu.SemaphoreType.DMA,
        pltpu.SemaphoreType.DMA,
        pltpu.SemaphoreType.DMA,
    ],
)
def _sc_gather(
    table_hbm, labels_hbm, out_hbm, idx_v, rows_v, table_sp, gsem, osem, isem,
    tsem,
):
    sid = lax.axis_index("s")
    wid = sid * NUM_CORES + lax.axis_index("c")
    base = wid * B_PER_W
    idx_c = pltpu.async_copy(labels_hbm.at[pl.ds(base, B_PER_W)], idx_v, isem)
    slab = pl.multiple_of(
        lax.min(sid * LOAD_ROWS, (NUM_SUBCORES - 1) * LOAD_ROWS - 24), 8
    )
    tab_c = pltpu.async_copy(
        table_hbm.at[pl.ds(slab, LOAD_ROWS)],
        table_sp.at[pl.ds(slab, LOAD_ROWS)],
        tsem,
    )

    idx_c.wait()
    gathers = []
    for j in range(1):
        gathers.append(
            pltpu.async_copy(
                table_hbm.at[idx_v.at[pl.ds(j * IDX_CHUNK, IDX_CHUNK)]],
                rows_v.at[pl.ds(j * IDX_CHUNK, IDX_CHUNK)],
                gsem.at[j],
            )
        )

    @pl.when(sid == 0)
    def _load_last_row():
        pltpu.sync_copy(
            table_hbm.at[pl.ds(TABLE_ROWS - 1, 1)],
            table_sp.at[pl.ds(TABLE_ROWS - 1, 1)],
        )

    tab_c.wait()
    plsc.subcore_barrier()
    for j in range(1, N_CHUNKS):
        gathers.append(
            pltpu.async_copy(
                table_sp.at[idx_v.at[pl.ds(j * IDX_CHUNK, IDX_CHUNK)]],
                rows_v.at[pl.ds(j * IDX_CHUNK, IDX_CHUNK)],
                gsem.at[j],
            )
        )
    outs = []
    for j in range(N_CHUNKS):
        gathers[j].wait()
        outs.append(
            pltpu.async_copy(
                rows_v.at[pl.ds(j * IDX_CHUNK, IDX_CHUNK)],
                out_hbm.at[pl.ds(base + j * IDX_CHUNK, IDX_CHUNK)],
                osem,
            )
        )
    for c in outs:
        c.wait()


def kernel(labels, train, embedding_table):
    del train
    return _sc_gather(embedding_table, labels.astype(jnp.int32))
